# baseline (speedup 1.0000x reference)
import jax
import jax.numpy as jnp
from jax import lax
from jax.experimental import pallas as pl
from jax.experimental.pallas import tpu as pltpu
from jax.experimental.pallas import tpu_sc as plsc

_BATCH = 16384
_FEAT = 64
_NC, _NS, _L = 2, 16, 16
_NW = _NC * _NS
_BPW = _BATCH // _NW


def _center_loss_tec(feat_hbm, lab_hbm, cent_hbm, out_hbm,
                     idx_v, rows_v, feat_v, acc_v, sem):
    wid = lax.axis_index("s") * _NC + lax.axis_index("c")
    base = wid * _BPW
    pltpu.sync_copy(lab_hbm.at[pl.ds(base, _BPW)], idx_v)
    gather = pltpu.async_copy(cent_hbm.at[idx_v], rows_v, sem)
    pltpu.sync_copy(feat_hbm.at[pl.ds(base, _BPW)], feat_v)
    gather.wait()

    def body(i, accs):
        a0, a1, a2, a3 = accs
        d0 = feat_v[i, pl.ds(0, _L)] - rows_v[i, pl.ds(0, _L)]
        d1 = feat_v[i, pl.ds(_L, _L)] - rows_v[i, pl.ds(_L, _L)]
        d2 = feat_v[i, pl.ds(2 * _L, _L)] - rows_v[i, pl.ds(2 * _L, _L)]
        d3 = feat_v[i, pl.ds(3 * _L, _L)] - rows_v[i, pl.ds(3 * _L, _L)]
        return (a0 + d0 * d0, a1 + d1 * d1, a2 + d2 * d2, a3 + d3 * d3)

    z = jnp.zeros((_L,), jnp.float32)
    a0, a1, a2, a3 = lax.fori_loop(0, _BPW, body, (z, z, z, z))
    acc_v[...] = (a0 + a1) + (a2 + a3)
    pltpu.sync_copy(acc_v, out_hbm.at[wid])


def kernel(features, labels, centers):
    if labels.ndim > 1:
        labels = jnp.squeeze(labels, axis=-1)
    mesh = plsc.VectorSubcoreMesh(core_axis_name="c", subcore_axis_name="s")
    partials = pl.kernel(
        _center_loss_tec,
        out_type=jax.ShapeDtypeStruct((_NW, _L), jnp.float32),
        mesh=mesh,
        compiler_params=pltpu.CompilerParams(use_tc_tiling_on_sc=False),
        scratch_types=[
            pltpu.VMEM((_BPW,), jnp.int32),
            pltpu.VMEM((_BPW, _FEAT), jnp.float32),
            pltpu.VMEM((_BPW, _FEAT), jnp.float32),
            pltpu.VMEM((_L,), jnp.float32),
            pltpu.SemaphoreType.DMA,
        ],
    )(features, labels.astype(jnp.int32), centers)
    return (jnp.sum(partials) / (2.0 * _BATCH)).astype(jnp.float32)


# speedup vs baseline: 1.1073x; 1.1073x over previous
import jax
import jax.numpy as jnp
from jax import lax
from jax.experimental import pallas as pl
from jax.experimental.pallas import tpu as pltpu
from jax.experimental.pallas import tpu_sc as plsc

_BATCH = 16384
_FEAT = 64
_CLS = 100000
_NC, _NS, _L = 2, 16, 16
_NW = _NC * _NS
_RPW = _FEAT // _NW
_QMAX = 25088
_PHASES = ((0, 25088), (25088, 25088), (50176, 25088), (75264, 24704))
_TAIL_LO = 99840
_TAIL_START = 99968


def _center_loss_tec(feat_hbm, lab_hbm, cent_hbm, out_hbm,
                     idx16_v, lab_v, f_v, c_v, tail_v, acc_v, sem):
    wid = lax.axis_index("s") * _NC + lax.axis_index("c")
    j0 = wid * _RPW
    lanes = lax.iota(jnp.int32, _L)
    idx16_v[...] = jnp.full((_L,), j0, jnp.int32) + jnp.minimum(lanes, 1)
    rows = idx16_v.at[pl.ds(0, _RPW)]
    pltpu.async_copy(feat_hbm.at[rows], f_v, sem).wait()
    pltpu.sync_copy(lab_hbm, lab_v)

    z = jnp.zeros((_L,), jnp.float32)
    zf = jnp.zeros((_L,), jnp.float32)
    zr = jnp.zeros((_L,), jnp.int32)
    accs = (z, z)

    for lo, q in _PHASES:
        pltpu.async_copy(cent_hbm.at[rows, pl.ds(lo, q)],
                         c_v.at[:, pl.ds(0, q)], sem).wait()
        lo_v = jnp.full((_L,), lo, jnp.int32)

        def body(t, accs, q=q, lo_v=lo_v):
            a0, a1 = accs
            sl = pl.ds(t * _L, _L)
            idx = lab_v[sl] - lo_v
            m = (idx >= 0) & (idx < q)
            idx = jnp.clip(idx, 0, q - 1)
            g0 = plsc.load_gather(c_v, [zr, idx], mask=m)
            g1 = plsc.load_gather(c_v, [zr + 1, idx], mask=m)
            d0 = jnp.where(m, f_v[0, sl] - g0, zf)
            d1 = jnp.where(m, f_v[1, sl] - g1, zf)
            return (a0 + d0 * d0, a1 + d1 * d1)

        accs = lax.fori_loop(0, _BATCH // _L, body, accs)

    pltpu.async_copy(cent_hbm.at[rows, pl.ds(_TAIL_LO, 128)], tail_v,
                     sem).wait()
    tlo_v = jnp.full((_L,), _TAIL_LO, jnp.int32)
    tst_v = jnp.full((_L,), _TAIL_START, jnp.int32)

    def tail_body(t, accs):
        a0, a1 = accs
        sl = pl.ds(t * _L, _L)
        lab = lab_v[sl]
        m = lab >= tst_v
        idx = jnp.clip(lab - tlo_v, 0, 127)
        g0 = plsc.load_gather(tail_v, [zr, idx], mask=m)
        g1 = plsc.load_gather(tail_v, [zr + 1, idx], mask=m)
        d0 = jnp.where(m, f_v[0, sl] - g0, zf)
        d1 = jnp.where(m, f_v[1, sl] - g1, zf)
        return (a0 + d0 * d0, a1 + d1 * d1)

    accs = lax.fori_loop(0, _BATCH // _L, tail_body, accs)

    acc_v[...] = accs[0] + accs[1]
    pltpu.sync_copy(acc_v, out_hbm.at[wid])


def kernel(features, labels, centers):
    if labels.ndim > 1:
        labels = jnp.squeeze(labels, axis=-1)
    mesh = plsc.VectorSubcoreMesh(core_axis_name="c", subcore_axis_name="s")
    partials = pl.kernel(
        _center_loss_tec,
        out_type=jax.ShapeDtypeStruct((_NW, _L), jnp.float32),
        mesh=mesh,
        compiler_params=pltpu.CompilerParams(needs_layout_passes=False),
        scratch_types=[
            pltpu.VMEM((_L,), jnp.int32),
            pltpu.VMEM((_BATCH,), jnp.int32),
            pltpu.VMEM((_RPW, _BATCH), jnp.float32),
            pltpu.VMEM((_RPW, _QMAX), jnp.float32),
            pltpu.VMEM((_RPW, 128), jnp.float32),
            pltpu.VMEM((_L,), jnp.float32),
            pltpu.SemaphoreType.DMA,
        ],
    )(features.T, labels.astype(jnp.int32), centers.T)
    return (jnp.sum(partials) / (2.0 * _BATCH)).astype(jnp.float32)


# speedup vs baseline: 1.3105x; 1.1835x over previous
import jax
import jax.numpy as jnp
from jax import lax
from jax.experimental import pallas as pl
from jax.experimental.pallas import tpu as pltpu
from jax.experimental.pallas import tpu_sc as plsc

_BATCH = 16384
_FEAT = 64
_CLS = 100000
_NC, _NS, _L = 2, 16, 16
_NW = _NC * _NS
_RPW = _FEAT // _NW
_QMAX = 25088
_PHASES = ((0, 25088), (25088, 25088), (50176, 25088), (75264, 24704))
_TAIL_LO = 99840
_TAIL_START = 99968


def _center_loss_tec(feat_hbm, lab_hbm, cent_hbm, out_hbm,
                     idx16_v, lab_v, f_v, c_v, tail_v, acc_v, sem):
    wid = lax.axis_index("s") * _NC + lax.axis_index("c")
    j0 = wid * _RPW
    lanes = lax.iota(jnp.int32, _L)
    idx16_v[...] = jnp.full((_L,), j0, jnp.int32) + jnp.minimum(lanes, 1)
    rows = idx16_v.at[pl.ds(0, _RPW)]
    pltpu.async_copy(feat_hbm.at[rows], f_v, sem).wait()
    pltpu.sync_copy(lab_hbm, lab_v)

    z = jnp.zeros((_L,), jnp.float32)
    zf = jnp.zeros((_L,), jnp.float32)
    zr = jnp.zeros((_L,), jnp.int32)
    accs = (z, z)

    for lo, q in _PHASES:
        pltpu.async_copy(cent_hbm.at[rows, pl.ds(lo, q)],
                         c_v.at[:, pl.ds(0, q)], sem).wait()
        lo_v = jnp.full((_L,), lo, jnp.int32)

        def body(t, accs, q=q, lo_v=lo_v):
            a0, a1 = accs
            sl = pl.ds(t * _L, _L)
            idx = lab_v[sl] - lo_v
            m = (idx >= 0) & (idx < q)
            idx = jnp.clip(idx, 0, q - 1)
            g0 = plsc.load_gather(c_v, [zr, idx], mask=m)
            g1 = plsc.load_gather(c_v, [zr + 1, idx], mask=m)
            d0 = jnp.where(m, f_v[0, sl] - g0, zf)
            d1 = jnp.where(m, f_v[1, sl] - g1, zf)
            return (a0 + d0 * d0, a1 + d1 * d1)

        accs = lax.fori_loop(0, _BATCH // _L, body, accs, unroll=8)

    pltpu.async_copy(cent_hbm.at[rows, pl.ds(_TAIL_LO, 128)], tail_v,
                     sem).wait()
    tlo_v = jnp.full((_L,), _TAIL_LO, jnp.int32)
    tst_v = jnp.full((_L,), _TAIL_START, jnp.int32)

    def tail_body(t, accs):
        a0, a1 = accs
        sl = pl.ds(t * _L, _L)
        lab = lab_v[sl]
        m = lab >= tst_v
        idx = jnp.clip(lab - tlo_v, 0, 127)
        g0 = plsc.load_gather(tail_v, [zr, idx], mask=m)
        g1 = plsc.load_gather(tail_v, [zr + 1, idx], mask=m)
        d0 = jnp.where(m, f_v[0, sl] - g0, zf)
        d1 = jnp.where(m, f_v[1, sl] - g1, zf)
        return (a0 + d0 * d0, a1 + d1 * d1)

    accs = lax.fori_loop(0, _BATCH // _L, tail_body, accs, unroll=8)

    acc_v[...] = accs[0] + accs[1]
    pltpu.sync_copy(acc_v, out_hbm.at[wid])


def kernel(features, labels, centers):
    if labels.ndim > 1:
        labels = jnp.squeeze(labels, axis=-1)
    mesh = plsc.VectorSubcoreMesh(core_axis_name="c", subcore_axis_name="s")
    partials = pl.kernel(
        _center_loss_tec,
        out_type=jax.ShapeDtypeStruct((_NW, _L), jnp.float32),
        mesh=mesh,
        compiler_params=pltpu.CompilerParams(needs_layout_passes=False),
        scratch_types=[
            pltpu.VMEM((_L,), jnp.int32),
            pltpu.VMEM((_BATCH,), jnp.int32),
            pltpu.VMEM((_RPW, _BATCH), jnp.float32),
            pltpu.VMEM((_RPW, _QMAX), jnp.float32),
            pltpu.VMEM((_RPW, 128), jnp.float32),
            pltpu.VMEM((_L,), jnp.float32),
            pltpu.SemaphoreType.DMA,
        ],
    )(features.T, labels.astype(jnp.int32), centers.T)
    return (jnp.sum(partials) / (2.0 * _BATCH)).astype(jnp.float32)


# speedup vs baseline: 1.8543x; 1.4150x over previous
import jax
import jax.numpy as jnp
from jax import lax
from jax.experimental import pallas as pl
from jax.experimental.pallas import tpu as pltpu
from jax.experimental.pallas import tpu_sc as plsc

_BATCH = 16384
_FEAT = 64
_CLS = 100000
_CLS_ALIGNED = 99968
_NC, _NS, _L = 2, 16, 16
_NW = _NC * _NS
_RPW = _FEAT // _NW
_LCH = 2048
_NCH = _BATCH // _LCH


def _center_loss_tec(feat_hbm, lab_hbm, cent_hbm, out_hbm,
                     idx16_v, lab_v, f_v, c_v, tail_v, acc_v,
                     csem, fsem, lsem):
    wid = lax.axis_index("s") * _NC + lax.axis_index("c")
    j0 = wid * _RPW
    lanes = lax.iota(jnp.int32, _L)
    idx16_v[...] = jnp.full((_L,), j0, jnp.int32) + lax.shift_right_logical(lanes, 3)

    blk = (j0 // 8) * 8
    zr = jnp.zeros((_L,), jnp.int32)
    ca_v = jnp.full((_L,), _CLS_ALIGNED, jnp.int32)
    z = jnp.zeros((_L,), jnp.float32)
    acc = z

    for unit in range(_RPW):
        row = idx16_v.at[pl.ds(unit * 8, 1)]
        cmain = pltpu.async_copy(
            cent_hbm.at[row, pl.ds(0, _CLS_ALIGNED)],
            c_v.at[:, pl.ds(0, _CLS_ALIGNED)], csem)
        if unit == 0:
            ctail = pltpu.async_copy(
                cent_hbm.at[pl.ds(blk, 8), pl.ds(_CLS_ALIGNED,
                                                 _CLS - _CLS_ALIGNED)],
                tail_v, csem)
        fcp = pltpu.async_copy(feat_hbm.at[row], f_v, fsem)
        lcp0 = pltpu.async_copy(lab_hbm.at[pl.ds(0, _LCH)],
                                lab_v.at[0], lsem)
        fcp.wait()
        cmain.wait()
        if unit == 0:
            ctail.wait()
        r8_v = jnp.full((_L,), j0 - blk + unit, jnp.int32)

        for k in range(_NCH):
            if k == 0:
                lcp0.wait()
            if k + 1 < _NCH:
                lnext = pltpu.async_copy(
                    lab_hbm.at[pl.ds((k + 1) * _LCH, _LCH)],
                    lab_v.at[(k + 1) % 2], lsem)

            def body(t, acc, k=k, r8_v=r8_v):
                lab = lab_v[k % 2, pl.ds(t * _L, _L)]
                mm = lab < ca_v
                mt = lab >= ca_v
                g = plsc.load_gather(c_v, [zr, lab], mask=mm)
                gt = plsc.load_gather(tail_v, [r8_v, lab - ca_v], mask=mt)
                f = f_v[0, pl.ds(k * _LCH + t * _L, _L)]
                d = f - jnp.where(mm, g, gt)
                return acc + d * d

            acc = lax.fori_loop(0, _LCH // _L, body, acc, unroll=8)
            if k + 1 < _NCH:
                lnext.wait()

    acc_v[...] = acc
    pltpu.sync_copy(acc_v, out_hbm.at[wid])


def kernel(features, labels, centers):
    if labels.ndim > 1:
        labels = jnp.squeeze(labels, axis=-1)
    mesh = plsc.VectorSubcoreMesh(core_axis_name="c", subcore_axis_name="s")
    partials = pl.kernel(
        _center_loss_tec,
        out_type=jax.ShapeDtypeStruct((_NW, _L), jnp.float32),
        mesh=mesh,
        compiler_params=pltpu.CompilerParams(needs_layout_passes=False),
        scratch_types=[
            pltpu.VMEM((_L,), jnp.int32),
            pltpu.VMEM((2, _LCH), jnp.int32),
            pltpu.VMEM((1, _BATCH), jnp.float32),
            pltpu.VMEM((1, _CLS), jnp.float32),
            pltpu.VMEM((8, _CLS - _CLS_ALIGNED), jnp.float32),
            pltpu.VMEM((_L,), jnp.float32),
            pltpu.SemaphoreType.DMA,
            pltpu.SemaphoreType.DMA,
            pltpu.SemaphoreType.DMA,
        ],
    )(features.T, labels.astype(jnp.int32), centers.T)
    return (jnp.sum(partials) / (2.0 * _BATCH)).astype(jnp.float32)
